# initial kernel scaffold (unmeasured)
import jax
import jax.numpy as jnp
from jax import lax
from jax.experimental import pallas as pl
from jax.experimental.pallas import tpu as pltpu


def kernel(
    x,
):
    def body(*refs):
        pass

    out_shape = jax.ShapeDtypeStruct(..., jnp.float32)
    return pl.pallas_call(body, out_shape=out_shape)(...)


# baseline (device time: 84828 ns/iter reference)
import jax
import jax.numpy as jnp
from jax import lax
from jax.experimental import pallas as pl
from jax.experimental.pallas import tpu as pltpu

N_DEV = 4


def kernel(x):
    m, n = x.shape
    ch = m // N_DEV

    def body(x_ref, out_ref, comm_ref, rs_send_sems, rs_recv_sems,
             ag_send_sems, ag_recv_sems):
        my = lax.axis_index("i")
        left = lax.rem(my + N_DEV - 1, N_DEV)
        right = lax.rem(my + 1, N_DEV)

        barrier_sem = pltpu.get_barrier_semaphore()
        for nbr in (left, right):
            pl.semaphore_signal(
                barrier_sem, inc=1,
                device_id=(nbr,), device_id_type=pl.DeviceIdType.MESH,
            )
        pl.semaphore_wait(barrier_sem, 2)

        out_ref[:, :] = x_ref[:, :]

        for s in range(N_DEV - 1):
            send_c = lax.rem(my + N_DEV - s, N_DEV)
            recv_c = lax.rem(my + N_DEV - s - 1, N_DEV)
            rdma = pltpu.make_async_remote_copy(
                src_ref=out_ref.at[pl.ds(send_c * ch, ch), :],
                dst_ref=comm_ref.at[s],
                send_sem=rs_send_sems.at[s],
                recv_sem=rs_recv_sems.at[s],
                device_id=(right,),
                device_id_type=pl.DeviceIdType.MESH,
            )
            rdma.start()
            rdma.wait()
            rows = pl.ds(recv_c * ch, ch)
            out_ref[rows, :] = out_ref[rows, :] + comm_ref[s]

        for s in range(N_DEV - 1):
            c = lax.rem(my + N_DEV + 1 - s, N_DEV)
            rows = pl.ds(c * ch, ch)
            rdma = pltpu.make_async_remote_copy(
                src_ref=out_ref.at[rows, :],
                dst_ref=out_ref.at[rows, :],
                send_sem=ag_send_sems.at[s],
                recv_sem=ag_recv_sems.at[s],
                device_id=(right,),
                device_id_type=pl.DeviceIdType.MESH,
            )
            rdma.start()
            rdma.wait()

    out_shape = jax.ShapeDtypeStruct((m, n), x.dtype)
    return pl.pallas_call(
        body,
        out_shape=out_shape,
        in_specs=[pl.BlockSpec(memory_space=pltpu.VMEM)],
        out_specs=pl.BlockSpec(memory_space=pltpu.VMEM),
        scratch_shapes=[
            pltpu.VMEM((N_DEV - 1, ch, n), x.dtype),
            pltpu.SemaphoreType.DMA((N_DEV - 1,)),
            pltpu.SemaphoreType.DMA((N_DEV - 1,)),
            pltpu.SemaphoreType.DMA((N_DEV - 1,)),
            pltpu.SemaphoreType.DMA((N_DEV - 1,)),
        ],
        compiler_params=pltpu.CompilerParams(collective_id=0),
    )(x)


# device time: 46770 ns/iter; 1.8137x vs baseline; 1.8137x over previous
import jax
import jax.numpy as jnp
from jax import lax
from jax.experimental import pallas as pl
from jax.experimental.pallas import tpu as pltpu

N_DEV = 4


def kernel(x):
    m, n = x.shape
    m2 = m // 2
    n2 = n // 2

    def body(x_ref, out_ref, comm_ref, send_sems, recv_sems):
        my = lax.axis_index("i")
        p1 = my + 1 - 2 * lax.rem(my, 2)
        p2 = 3 - my

        barrier_sem = pltpu.get_barrier_semaphore()
        for nbr in (p1, p2):
            pl.semaphore_signal(
                barrier_sem, inc=1,
                device_id=(nbr,), device_id_type=pl.DeviceIdType.MESH,
            )
        pl.semaphore_wait(barrier_sem, 2)

        out_ref[:, :] = x_ref[:, :]

        h_a = jnp.where(jnp.logical_or(my == 0, my == 3), 0, m2)
        h_b = jnp.where(my < 2, 0, m2)
        halves = [h_a, h_b]
        partners = [(p1, p2, p1), (p2, p1, p2)]
        col0 = [0, n2]

        def exchange(b, stage, src_rows, dst_is_out, dst_rows):
            cols = pl.ds(col0[b], n2)
            dst = (out_ref.at[dst_rows, cols] if dst_is_out
                   else comm_ref.at[2 * b + stage])
            return pltpu.make_async_remote_copy(
                src_ref=out_ref.at[src_rows, cols],
                dst_ref=dst,
                send_sem=send_sems.at[3 * b + stage],
                recv_sem=recv_sems.at[3 * b + stage],
                device_id=(partners[b][stage],),
                device_id_type=pl.DeviceIdType.MESH,
            )

        for stage in range(2):
            rdmas = []
            for b in range(2):
                h = halves[b]
                src_rows = pl.ds(m2 - h if stage == 0 else h, m2)
                rdma = exchange(b, stage, src_rows, False, None)
                rdma.start()
                rdmas.append(rdma)
            for b in range(2):
                rdmas[b].wait()
                rows = pl.ds(halves[b], m2)
                cols = pl.ds(col0[b], n2)
                out_ref[rows, cols] = out_ref[rows, cols] + comm_ref[2 * b + stage]

        rdmas = []
        for b in range(2):
            rows = pl.ds(halves[b], m2)
            rdma = exchange(b, 2, rows, True, rows)
            rdma.start()
            rdmas.append(rdma)
        for b in range(2):
            rdmas[b].wait()

    out_shape = jax.ShapeDtypeStruct((m, n), x.dtype)
    return pl.pallas_call(
        body,
        out_shape=out_shape,
        in_specs=[pl.BlockSpec(memory_space=pltpu.VMEM)],
        out_specs=pl.BlockSpec(memory_space=pltpu.VMEM),
        scratch_shapes=[
            pltpu.VMEM((4, m2, n2), x.dtype),
            pltpu.SemaphoreType.DMA((6,)),
            pltpu.SemaphoreType.DMA((6,)),
        ],
        compiler_params=pltpu.CompilerParams(collective_id=0),
    )(x)


# device time: 46361 ns/iter; 1.8297x vs baseline; 1.0088x over previous
import jax
import jax.numpy as jnp
from jax import lax
from jax.experimental import pallas as pl
from jax.experimental.pallas import tpu as pltpu

N_DEV = 4


def kernel(x):
    m, n = x.shape
    m2 = m // 2
    mh = m // 4

    def body(x_ref, out_ref, comm_ref, send_sems, recv_sems):
        my = lax.axis_index("i")
        p1 = my + 1 - 2 * lax.rem(my, 2)
        p2 = 3 - my

        barrier_sem = pltpu.get_barrier_semaphore()
        for nbr in (p1, p2):
            pl.semaphore_signal(
                barrier_sem, inc=1,
                device_id=(nbr,), device_id_type=pl.DeviceIdType.MESH,
            )
        pl.semaphore_wait(barrier_sem, 2)

        h_a = jnp.where(jnp.logical_or(my == 0, my == 3), 0, mh)
        h_b = jnp.where(my < 2, 0, mh)
        own = [0 * m2 + h_a, 1 * m2 + h_b]
        oth = [0 * m2 + (mh - h_a), 1 * m2 + (mh - h_b)]
        partners = [(p1, p2, p1), (p2, p1, p2)]

        def exchange(b, stage, src_ref, dst_ref):
            return pltpu.make_async_remote_copy(
                src_ref=src_ref,
                dst_ref=dst_ref,
                send_sem=send_sems.at[3 * b + stage],
                recv_sem=recv_sems.at[3 * b + stage],
                device_id=(partners[b][stage],),
                device_id_type=pl.DeviceIdType.MESH,
            )

        s1 = [exchange(b, 0, x_ref.at[pl.ds(oth[b], mh), :], comm_ref.at[2 * b])
              for b in range(2)]
        for r in s1:
            r.start()

        s2 = [None, None]
        for b in range(2):
            s1[b].wait()
            rows = pl.ds(own[b], mh)
            out_ref[rows, :] = x_ref[rows, :] + comm_ref[2 * b]
            s2[b] = exchange(b, 1, out_ref.at[rows, :], comm_ref.at[2 * b + 1])
            s2[b].start()

        s3 = [None, None]
        for b in range(2):
            s2[b].wait()
            rows = pl.ds(own[b], mh)
            out_ref[rows, :] = out_ref[rows, :] + comm_ref[2 * b + 1]
            s3[b] = exchange(b, 2, out_ref.at[rows, :], out_ref.at[rows, :])
            s3[b].start()

        for b in range(2):
            s3[b].wait()

    out_shape = jax.ShapeDtypeStruct((m, n), x.dtype)
    return pl.pallas_call(
        body,
        out_shape=out_shape,
        in_specs=[pl.BlockSpec(memory_space=pltpu.VMEM)],
        out_specs=pl.BlockSpec(memory_space=pltpu.VMEM),
        scratch_shapes=[
            pltpu.VMEM((4, mh, n), x.dtype),
            pltpu.SemaphoreType.DMA((6,)),
            pltpu.SemaphoreType.DMA((6,)),
        ],
        compiler_params=pltpu.CompilerParams(collective_id=0),
    )(x)


# device time: 43047 ns/iter; 1.9706x vs baseline; 1.0770x over previous
import jax
import jax.numpy as jnp
from jax import lax
from jax.experimental import pallas as pl
from jax.experimental.pallas import tpu as pltpu

N_DEV = 4
Q = 4


def kernel(x):
    m, n = x.shape
    m2 = m // 2
    mh = m // 4
    qh = mh // Q

    def body(x_ref, out_ref, comm_ref, send_sems, recv_sems):
        my = lax.axis_index("i")
        p1 = my + 1 - 2 * lax.rem(my, 2)
        p2 = 3 - my

        barrier_sem = pltpu.get_barrier_semaphore()
        for nbr in (p1, p2):
            pl.semaphore_signal(
                barrier_sem, inc=1,
                device_id=(nbr,), device_id_type=pl.DeviceIdType.MESH,
            )
        pl.semaphore_wait(barrier_sem, 2)

        h_a = jnp.where(jnp.logical_or(my == 0, my == 3), 0, mh)
        h_b = jnp.where(my < 2, 0, mh)
        own = [0 * m2 + h_a, 1 * m2 + h_b]
        oth = [0 * m2 + (mh - h_a), 1 * m2 + (mh - h_b)]
        partners = [(p1, p2, p1), (p2, p1, p2)]

        def sem_idx(b, stage, q):
            return (b * 3 + stage) * Q + q

        def exchange(b, stage, q, src_ref, dst_ref):
            return pltpu.make_async_remote_copy(
                src_ref=src_ref,
                dst_ref=dst_ref,
                send_sem=send_sems.at[sem_idx(b, stage, q)],
                recv_sem=recv_sems.at[sem_idx(b, stage, q)],
                device_id=(partners[b][stage],),
                device_id_type=pl.DeviceIdType.MESH,
            )

        s1 = [[None] * Q, [None] * Q]
        for q in range(Q):
            for b in range(2):
                r = exchange(
                    b, 0, q,
                    x_ref.at[pl.ds(oth[b] + q * qh, qh), :],
                    comm_ref.at[2 * b, pl.ds(q * qh, qh), :],
                )
                r.start()
                s1[b][q] = r

        s2 = [[None] * Q, [None] * Q]
        for q in range(Q):
            for b in range(2):
                s1[b][q].wait()
                rows = pl.ds(own[b] + q * qh, qh)
                crows = pl.ds(q * qh, qh)
                out_ref[rows, :] = x_ref[rows, :] + comm_ref[2 * b, crows, :]
                r = exchange(b, 1, q, out_ref.at[rows, :],
                             comm_ref.at[2 * b + 1, crows, :])
                r.start()
                s2[b][q] = r

        s3 = [[None] * Q, [None] * Q]
        for q in range(Q):
            for b in range(2):
                s2[b][q].wait()
                rows = pl.ds(own[b] + q * qh, qh)
                crows = pl.ds(q * qh, qh)
                out_ref[rows, :] = out_ref[rows, :] + comm_ref[2 * b + 1, crows, :]
                r = exchange(b, 2, q, out_ref.at[rows, :], out_ref.at[rows, :])
                r.start()
                s3[b][q] = r

        for q in range(Q):
            for b in range(2):
                s3[b][q].wait()

    out_shape = jax.ShapeDtypeStruct((m, n), x.dtype)
    return pl.pallas_call(
        body,
        out_shape=out_shape,
        in_specs=[pl.BlockSpec(memory_space=pltpu.VMEM)],
        out_specs=pl.BlockSpec(memory_space=pltpu.VMEM),
        scratch_shapes=[
            pltpu.VMEM((4, mh, n), x.dtype),
            pltpu.SemaphoreType.DMA((6 * Q,)),
            pltpu.SemaphoreType.DMA((6 * Q,)),
        ],
        compiler_params=pltpu.CompilerParams(collective_id=0),
    )(x)
